# baseline (speedup 1.0000x reference)
import functools

import jax
import jax.numpy as jnp
from jax import lax
from jax.experimental import pallas as pl
from jax.experimental.pallas import tpu as pltpu
from jax.experimental.pallas import tpu_sc as plsc

_N = 10000
_E = 320000
_D = 128
_G = 256
_NS = 16
_N_PAD = 10240
_EPT = _E // _NS
_CH = 80
_NCHT = _EPT // _CH
_SECC = 250
_CHS = 40
_NCHS = _EPT // _CHS
_SECS = 500
_ROWS_PT = _N_PAD // _NS
_RB = 1000
_NBLK = _N // _RB
_NSTEP = 2 * _NBLK

_f32 = jnp.float32
_bf16 = jnp.bfloat16
_i32 = jnp.int32


def _deg_body(dst_hbm, out_hbm, dst_v, deg_v, tmp_v, acc_v, deg_sp):
    c = lax.axis_index("c")
    s = lax.axis_index("s")
    seg = _N_PAD // _NS
    zeros16 = jnp.zeros((16,), _f32)
    ones16 = jnp.full((16,), 1.0, _f32)

    def zbody(i, carry):
        deg_v[pl.ds(i * 16, 16)] = zeros16
        return carry

    lax.fori_loop(0, _N_PAD // 16, zbody, 0)
    pltpu.sync_copy(dst_hbm.at[c, s], dst_v)

    def body(i, carry):
        for k in range(_CH // 16):
            idx = dst_v[i, pl.ds(k * 16, 16)]
            plsc.addupdate_scatter(deg_v, [idx], ones16)
        return carry

    lax.fori_loop(0, _NCHT, body, 0)
    pltpu.sync_copy(deg_v, deg_sp.at[s])
    plsc.subcore_barrier()

    def azero(i, carry):
        acc_v[pl.ds(i * 16, 16)] = zeros16
        return carry

    lax.fori_loop(0, seg // 16, azero, 0)
    for t in range(_NS):
        pltpu.sync_copy(deg_sp.at[t, pl.ds(s * seg, seg)], tmp_v)

        def radd(i, carry):
            acc_v[pl.ds(i * 16, 16)] += tmp_v[pl.ds(i * 16, 16)]
            return carry

        lax.fori_loop(0, seg // 16, radd, 0)
    pltpu.sync_copy(acc_v, out_hbm.at[c, pl.ds(s * seg, seg)])


def _spmm_body(src_hbm, dst_hbm, hn_hbm, zeros_hbm, out_hbm,
               src_v, dst_v, buf0, buf1, acc_sp, sem0, sem1):
    c = lax.axis_index("c")
    s = lax.axis_index("s")
    base = s * _ROWS_PT
    pltpu.sync_copy(zeros_hbm, buf0)
    for k in range(_ROWS_PT // _CHS):
        pltpu.sync_copy(buf0, acc_sp.at[pl.ds(base + k * _CHS, _CHS)])
    plsc.subcore_barrier()

    def section(sec, carry):
        pltpu.sync_copy(src_hbm.at[c, s, pl.ds(sec * _SECS, _SECS)], src_v)
        pltpu.sync_copy(dst_hbm.at[c, s, pl.ds(sec * _SECS, _SECS)], dst_v)
        pltpu.async_copy(hn_hbm.at[src_v.at[0]], buf0, sem0)

        def body(g, carry2):
            j0 = 2 * g
            j1 = j0 + 1
            pltpu.async_copy(hn_hbm.at[src_v.at[j1]], buf1, sem1)
            pltpu.make_async_copy(hn_hbm.at[src_v.at[j0]], buf0, sem0).wait()
            pltpu.sync_copy(buf0, acc_sp.at[dst_v.at[j0]], add=True)

            @pl.when(g < _SECS // 2 - 1)
            def _():
                pltpu.async_copy(hn_hbm.at[src_v.at[j0 + 2]], buf0, sem0)

            pltpu.make_async_copy(hn_hbm.at[src_v.at[j1]], buf1, sem1).wait()
            pltpu.sync_copy(buf1, acc_sp.at[dst_v.at[j1]], add=True)
            return carry2

        lax.fori_loop(0, _SECS // 2, body, 0)
        return carry

    lax.fori_loop(0, _NCHS // _SECS, section, 0)
    plsc.subcore_barrier()
    for k in range(_ROWS_PT // _CHS):
        r = base + k * _CHS
        pltpu.sync_copy(acc_sp.at[pl.ds(r, _CHS)], buf0)
        pltpu.sync_copy(buf0, out_hbm.at[c, pl.ds(r, _CHS)])


@functools.lru_cache(maxsize=1)
def _sc_kernels():
    mesh = plsc.VectorSubcoreMesh(core_axis_name="c", subcore_axis_name="s")
    deg_k = pl.kernel(
        _deg_body,
        out_type=jax.ShapeDtypeStruct((2, _N_PAD), _f32),
        mesh=mesh,
        compiler_params=pltpu.CompilerParams(needs_layout_passes=False),
        scratch_types=[
            pltpu.VMEM((_NCHT, _CH), _i32),
            pltpu.VMEM((_N_PAD,), _f32),
            pltpu.VMEM((_N_PAD // _NS,), _f32),
            pltpu.VMEM((_N_PAD // _NS,), _f32),
            pltpu.VMEM_SHARED((_NS, _N_PAD), _f32),
        ],
    )
    spmm_k = pl.kernel(
        _spmm_body,
        out_type=jax.ShapeDtypeStruct((2, _N_PAD, 128), _bf16),
        mesh=mesh,
        compiler_params=pltpu.CompilerParams(use_tc_tiling_on_sc=False),
        scratch_types=[
            pltpu.VMEM((_SECS, _CHS), _i32),
            pltpu.VMEM((_SECS, _CHS), _i32),
            pltpu.VMEM((_CHS, 128), _bf16),
            pltpu.VMEM((_CHS, 128), _bf16),
            pltpu.VMEM_SHARED((_N_PAD, 128), _bf16),
            pltpu.SemaphoreType.DMA,
            pltpu.SemaphoreType.DMA,
        ],
    )
    return deg_k, spmm_k


def _hn_body(x_ref, w_ref, b_ref, deg_ref, hn16_ref):
    h = jnp.dot(x_ref[...].astype(_bf16), w_ref[...].astype(_bf16),
                preferred_element_type=_f32)
    h = h + b_ref[...]
    dcol = jnp.transpose(deg_ref[0], (1, 0))
    rs = lax.rsqrt(dcol + 1.0)
    hn16_ref[...] = (h * rs).astype(_bf16)


def _hn_call(x_all, w, b, deg):
    grid = (2 * _N) // _RB
    return pl.pallas_call(
        _hn_body,
        grid=(grid,),
        in_specs=[
            pl.BlockSpec((_RB, _D), lambda i: (i, 0)),
            pl.BlockSpec((_D, _D), lambda i: (0, 0)),
            pl.BlockSpec((1, _D), lambda i: (0, 0)),
            pl.BlockSpec((1, 1, _RB), lambda i: (i, 0, 0)),
        ],
        out_specs=pl.BlockSpec((_RB, _D), lambda i: (i, 0)),
        out_shape=jax.ShapeDtypeStruct((2 * _N, _D), _bf16),
    )(x_all, w, b, deg)


def _pool_body(s_ref, hn_ref, deg_ref, batch_ref, wl_ref, bl_ref, out_ref,
               pooled, cnts):
    i = pl.program_id(0)

    @pl.when(i == 0)
    def _():
        pooled[...] = jnp.zeros_like(pooled)
        cnts[...] = jnp.zeros_like(cnts)

    s32 = s_ref[0].astype(_f32)
    h32 = hn_ref[...].astype(_f32)
    dcol = jnp.transpose(deg_ref[0], (1, 0))
    rs = lax.rsqrt(dcol + 1.0)
    agg = jnp.maximum(rs * (s32 + h32), 0.0).astype(_bf16)
    b = batch_ref[0, 0, :]
    onehot = (b[:, None] == lax.broadcasted_iota(_i32, (_RB, _G), 1))
    onehot = onehot.astype(_bf16)
    psum = lax.dot_general(onehot, agg, (((0,), (0,)), ((), ())),
                           preferred_element_type=_f32)
    csum = lax.dot_general(onehot, jnp.ones((_RB, 8), _bf16),
                           (((0,), (0,)), ((), ())),
                           preferred_element_type=_f32)

    @pl.when(i < _NBLK)
    def _():
        pooled[0] += psum
        cnts[0] += csum

    @pl.when(i >= _NBLK)
    def _():
        pooled[1] += psum
        cnts[1] += csum

    @pl.when(i == _NSTEP - 1)
    def _():
        ca = jnp.maximum(cnts[0][:, 0:1], 1.0)
        cb = jnp.maximum(cnts[1][:, 0:1], 1.0)
        fa = pooled[0] / ca
        fb = pooled[1] / cb
        wl = wl_ref[...]
        out_ref[...] = (
            jnp.dot(fa + fb, wl[0:_D], preferred_element_type=_f32)
            + jnp.dot(jnp.minimum(fa, fb), wl[_D:], preferred_element_type=_f32)
            + bl_ref[...]
        )


def _pool_call(s_all, hn16, deg3d, batch3d, w_lin, b_lin):
    return pl.pallas_call(
        _pool_body,
        grid=(_NSTEP,),
        in_specs=[
            pl.BlockSpec((1, _RB, _D), lambda i: (i // _NBLK, i % _NBLK, 0)),
            pl.BlockSpec((_RB, _D), lambda i: (i, 0)),
            pl.BlockSpec((1, 1, _RB), lambda i: (i, 0, 0)),
            pl.BlockSpec((1, 1, _RB), lambda i: (i, 0, 0)),
            pl.BlockSpec((2 * _D, _D), lambda i: (0, 0)),
            pl.BlockSpec((1, _D), lambda i: (0, 0)),
        ],
        out_specs=pl.BlockSpec((_G, _D), lambda i: (0, 0)),
        out_shape=jax.ShapeDtypeStruct((_G, _D), _f32),
        scratch_shapes=[
            pltpu.VMEM((2, _G, _D), _f32),
            pltpu.VMEM((2, _G, 8), _f32),
        ],
    )(s_all, hn16, deg3d, batch3d, w_lin, b_lin)


def kernel(druga_x, druga_edge_index, druga_batch,
           drugb_x, drugb_edge_index, drugb_batch,
           W_gcn, b_gcn, W_lin, b_lin):
    ea = druga_edge_index.astype(_i32)
    eb = drugb_edge_index.astype(_i32)
    src_idx = jnp.stack([ea[0], eb[0] + _N]).reshape(2, _NS, _NCHT, _CH)
    dst_idx = jnp.stack([ea[1], eb[1]]).reshape(2, _NS, _NCHT, _CH)
    zeros16 = jnp.zeros((_CHS, 128), _bf16)

    deg_kernel, spmm_kernel = _sc_kernels()
    deg2 = deg_kernel(dst_idx)
    deg3d = deg2[:, :_N].reshape(_NSTEP, 1, _RB)

    x_all = jnp.concatenate([druga_x, drugb_x])
    hn16 = _hn_call(x_all, W_gcn, b_gcn.reshape(1, _D), deg3d)

    spmm = spmm_kernel(src_idx.reshape(2, _NS, _NCHS, _CHS),
                       dst_idx.reshape(2, _NS, _NCHS, _CHS),
                       hn16, zeros16)

    batch3d = jnp.stack([druga_batch.astype(_i32),
                         drugb_batch.astype(_i32)]).reshape(_NSTEP, 1, _RB)

    return _pool_call(spmm, hn16, deg3d, batch3d, W_lin,
                      b_lin.reshape(1, _D))


# speedup vs baseline: 1.3106x; 1.3106x over previous
import functools

import jax
import jax.numpy as jnp
from jax import lax
from jax.experimental import pallas as pl
from jax.experimental.pallas import tpu as pltpu
from jax.experimental.pallas import tpu_sc as plsc

_N = 10000
_E = 320000
_D = 128
_G = 256
_NS = 16
_N_PAD = 10240
_EPT = _E // _NS
_CH = 80
_NCHT = _EPT // _CH
_SECC = 250
_CHS = 80
_NCHS = _EPT // _CHS
_SECS = 250
_ROWS_PT = _N_PAD // _NS
_RB = 1000
_NBLK = _N // _RB
_NSTEP = 2 * _NBLK

_f32 = jnp.float32
_bf16 = jnp.bfloat16
_i32 = jnp.int32


def _deg_body(dst_hbm, out_hbm, dst_v, deg_v, tmp_v, acc_v, deg_sp):
    c = lax.axis_index("c")
    s = lax.axis_index("s")
    seg = _N_PAD // _NS
    zeros16 = jnp.zeros((16,), _f32)
    ones16 = jnp.full((16,), 1.0, _f32)

    def zbody(i, carry):
        deg_v[pl.ds(i * 16, 16)] = zeros16
        return carry

    lax.fori_loop(0, _N_PAD // 16, zbody, 0)
    pltpu.sync_copy(dst_hbm.at[c, s], dst_v)

    def body(i, carry):
        for k in range(_CH // 16):
            idx = dst_v[i, pl.ds(k * 16, 16)]
            plsc.addupdate_scatter(deg_v, [idx], ones16)
        return carry

    lax.fori_loop(0, _NCHT, body, 0)
    pltpu.sync_copy(deg_v, deg_sp.at[s])
    plsc.subcore_barrier()

    def azero(i, carry):
        acc_v[pl.ds(i * 16, 16)] = zeros16
        return carry

    lax.fori_loop(0, seg // 16, azero, 0)
    for t in range(_NS):
        pltpu.sync_copy(deg_sp.at[t, pl.ds(s * seg, seg)], tmp_v)

        def radd(i, carry):
            acc_v[pl.ds(i * 16, 16)] += tmp_v[pl.ds(i * 16, 16)]
            return carry

        lax.fori_loop(0, seg // 16, radd, 0)
    pltpu.sync_copy(acc_v, out_hbm.at[c, pl.ds(s * seg, seg)])


def _spmm_body(src_hbm, dst_hbm, hn_hbm, zeros_hbm, out_hbm,
               src_v, dst_v, buf0, buf1, acc_sp, sem0, sem1):
    c = lax.axis_index("c")
    s = lax.axis_index("s")
    base = s * _ROWS_PT
    pltpu.sync_copy(zeros_hbm, buf0)
    for k in range(_ROWS_PT // _CHS):
        pltpu.sync_copy(buf0, acc_sp.at[pl.ds(base + k * _CHS, _CHS)])
    plsc.subcore_barrier()

    def section(sec, carry):
        pltpu.sync_copy(src_hbm.at[c, s, pl.ds(sec * _SECS, _SECS)], src_v)
        pltpu.sync_copy(dst_hbm.at[c, s, pl.ds(sec * _SECS, _SECS)], dst_v)
        pltpu.async_copy(hn_hbm.at[src_v.at[0]], buf0, sem0)

        def body(g, carry2):
            j0 = 2 * g
            j1 = j0 + 1
            pltpu.async_copy(hn_hbm.at[src_v.at[j1]], buf1, sem1)
            pltpu.make_async_copy(hn_hbm.at[src_v.at[j0]], buf0, sem0).wait()
            pltpu.sync_copy(buf0, acc_sp.at[dst_v.at[j0]], add=True)

            @pl.when(g < _SECS // 2 - 1)
            def _():
                pltpu.async_copy(hn_hbm.at[src_v.at[j0 + 2]], buf0, sem0)

            pltpu.make_async_copy(hn_hbm.at[src_v.at[j1]], buf1, sem1).wait()
            pltpu.sync_copy(buf1, acc_sp.at[dst_v.at[j1]], add=True)
            return carry2

        lax.fori_loop(0, _SECS // 2, body, 0)
        return carry

    lax.fori_loop(0, _NCHS // _SECS, section, 0)
    plsc.subcore_barrier()
    for k in range(_ROWS_PT // _CHS):
        r = base + k * _CHS
        pltpu.sync_copy(acc_sp.at[pl.ds(r, _CHS)], buf0)
        pltpu.sync_copy(buf0, out_hbm.at[c, pl.ds(r, _CHS)])


@functools.lru_cache(maxsize=1)
def _sc_kernels():
    mesh = plsc.VectorSubcoreMesh(core_axis_name="c", subcore_axis_name="s")
    deg_k = pl.kernel(
        _deg_body,
        out_type=jax.ShapeDtypeStruct((2, _N_PAD), _f32),
        mesh=mesh,
        compiler_params=pltpu.CompilerParams(needs_layout_passes=False),
        scratch_types=[
            pltpu.VMEM((_NCHT, _CH), _i32),
            pltpu.VMEM((_N_PAD,), _f32),
            pltpu.VMEM((_N_PAD // _NS,), _f32),
            pltpu.VMEM((_N_PAD // _NS,), _f32),
            pltpu.VMEM_SHARED((_NS, _N_PAD), _f32),
        ],
    )
    spmm_k = pl.kernel(
        _spmm_body,
        out_type=jax.ShapeDtypeStruct((2, _N_PAD, 128), _bf16),
        mesh=mesh,
        compiler_params=pltpu.CompilerParams(use_tc_tiling_on_sc=False),
        scratch_types=[
            pltpu.VMEM((_SECS, _CHS), _i32),
            pltpu.VMEM((_SECS, _CHS), _i32),
            pltpu.VMEM((_CHS, 128), _bf16),
            pltpu.VMEM((_CHS, 128), _bf16),
            pltpu.VMEM_SHARED((_N_PAD, 128), _bf16),
            pltpu.SemaphoreType.DMA,
            pltpu.SemaphoreType.DMA,
        ],
    )
    return deg_k, spmm_k


def _hn_body(x_ref, w_ref, b_ref, deg_ref, hn16_ref):
    h = jnp.dot(x_ref[...].astype(_bf16), w_ref[...].astype(_bf16),
                preferred_element_type=_f32)
    h = h + b_ref[...]
    dcol = jnp.transpose(deg_ref[0], (1, 0))
    rs = lax.rsqrt(dcol + 1.0)
    hn16_ref[...] = (h * rs).astype(_bf16)


def _hn_call(x_all, w, b, deg):
    grid = (2 * _N) // _RB
    return pl.pallas_call(
        _hn_body,
        grid=(grid,),
        in_specs=[
            pl.BlockSpec((_RB, _D), lambda i: (i, 0)),
            pl.BlockSpec((_D, _D), lambda i: (0, 0)),
            pl.BlockSpec((1, _D), lambda i: (0, 0)),
            pl.BlockSpec((1, 1, _RB), lambda i: (i, 0, 0)),
        ],
        out_specs=pl.BlockSpec((_RB, _D), lambda i: (i, 0)),
        out_shape=jax.ShapeDtypeStruct((2 * _N, _D), _bf16),
    )(x_all, w, b, deg)


def _pool_body(s_ref, hn_ref, deg_ref, batch_ref, wl_ref, bl_ref, out_ref,
               pooled, cnts):
    i = pl.program_id(0)

    @pl.when(i == 0)
    def _():
        pooled[...] = jnp.zeros_like(pooled)
        cnts[...] = jnp.zeros_like(cnts)

    s32 = s_ref[0].astype(_f32)
    h32 = hn_ref[...].astype(_f32)
    dcol = jnp.transpose(deg_ref[0], (1, 0))
    rs = lax.rsqrt(dcol + 1.0)
    agg = jnp.maximum(rs * (s32 + h32), 0.0).astype(_bf16)
    b = batch_ref[0, 0, :]
    onehot = (b[:, None] == lax.broadcasted_iota(_i32, (_RB, _G), 1))
    onehot = onehot.astype(_bf16)
    psum = lax.dot_general(onehot, agg, (((0,), (0,)), ((), ())),
                           preferred_element_type=_f32)
    csum = lax.dot_general(onehot, jnp.ones((_RB, 8), _bf16),
                           (((0,), (0,)), ((), ())),
                           preferred_element_type=_f32)

    @pl.when(i < _NBLK)
    def _():
        pooled[0] += psum
        cnts[0] += csum

    @pl.when(i >= _NBLK)
    def _():
        pooled[1] += psum
        cnts[1] += csum

    @pl.when(i == _NSTEP - 1)
    def _():
        ca = jnp.maximum(cnts[0][:, 0:1], 1.0)
        cb = jnp.maximum(cnts[1][:, 0:1], 1.0)
        fa = pooled[0] / ca
        fb = pooled[1] / cb
        wl = wl_ref[...]
        out_ref[...] = (
            jnp.dot(fa + fb, wl[0:_D], preferred_element_type=_f32)
            + jnp.dot(jnp.minimum(fa, fb), wl[_D:], preferred_element_type=_f32)
            + bl_ref[...]
        )


def _pool_call(s_all, hn16, deg3d, batch3d, w_lin, b_lin):
    return pl.pallas_call(
        _pool_body,
        grid=(_NSTEP,),
        in_specs=[
            pl.BlockSpec((1, _RB, _D), lambda i: (i // _NBLK, i % _NBLK, 0)),
            pl.BlockSpec((_RB, _D), lambda i: (i, 0)),
            pl.BlockSpec((1, 1, _RB), lambda i: (i, 0, 0)),
            pl.BlockSpec((1, 1, _RB), lambda i: (i, 0, 0)),
            pl.BlockSpec((2 * _D, _D), lambda i: (0, 0)),
            pl.BlockSpec((1, _D), lambda i: (0, 0)),
        ],
        out_specs=pl.BlockSpec((_G, _D), lambda i: (0, 0)),
        out_shape=jax.ShapeDtypeStruct((_G, _D), _f32),
        scratch_shapes=[
            pltpu.VMEM((2, _G, _D), _f32),
            pltpu.VMEM((2, _G, 8), _f32),
        ],
    )(s_all, hn16, deg3d, batch3d, w_lin, b_lin)


def kernel(druga_x, druga_edge_index, druga_batch,
           drugb_x, drugb_edge_index, drugb_batch,
           W_gcn, b_gcn, W_lin, b_lin):
    ea = druga_edge_index.astype(_i32)
    eb = drugb_edge_index.astype(_i32)
    src_idx = jnp.stack([ea[0], eb[0] + _N]).reshape(2, _NS, _NCHT, _CH)
    dst_idx = jnp.stack([ea[1], eb[1]]).reshape(2, _NS, _NCHT, _CH)
    zeros16 = jnp.zeros((_CHS, 128), _bf16)

    deg_kernel, spmm_kernel = _sc_kernels()
    deg2 = deg_kernel(dst_idx)
    deg3d = deg2[:, :_N].reshape(_NSTEP, 1, _RB)

    x_all = jnp.concatenate([druga_x, drugb_x])
    hn16 = _hn_call(x_all, W_gcn, b_gcn.reshape(1, _D), deg3d)

    spmm = spmm_kernel(src_idx.reshape(2, _NS, _NCHS, _CHS),
                       dst_idx.reshape(2, _NS, _NCHS, _CHS),
                       hn16, zeros16)

    batch3d = jnp.stack([druga_batch.astype(_i32),
                         drugb_batch.astype(_i32)]).reshape(_NSTEP, 1, _RB)

    return _pool_call(spmm, hn16, deg3d, batch3d, W_lin,
                      b_lin.reshape(1, _D))
